# Initial kernel scaffold; baseline (speedup 1.0000x reference)
import jax
import jax.numpy as jnp
from jax.experimental import pallas as pl


def kernel(d_tx, d_rx, apod, rf, t0):
    raise NotImplementedError("write your pallas kernel here")


# speedup vs baseline: 940.7584x; 940.7584x over previous
import functools

import jax
import jax.numpy as jnp
from jax import lax
from jax.experimental import pallas as pl
from jax.experimental.pallas import tpu as pltpu
from jax.experimental.pallas import tpu_sc as plsc

N_ANGLES, N_ELEMENTS, N_SAMP = 16, 128, 2048
NZ, NX = 512, 256
NPIX = NZ * NX
FS = float(N_SAMP) / 2.0

N_TILES = 32
PIX_PER_TILE = NPIX // N_TILES
S_PAD = N_SAMP + 8
LANES = 16
VECS = PIX_PER_TILE // LANES


def _das_body(dtx_hbm, drx_hbm, apod_hbm, rf_hbm, t0_hbm, out_hbm,
              dtx_v, drx_v, apod_v, rf_v, acc_v, t0_v):
    wid = lax.axis_index("c") * 16 + lax.axis_index("s")
    base = wid * PIX_PER_TILE

    pltpu.sync_copy(t0_hbm, t0_v)

    def angle_body(a, _):
        pltpu.sync_copy(dtx_hbm.at[a, pl.ds(base, PIX_PER_TILE)], dtx_v)
        t0_b = plsc.load_gather(t0_v, [jnp.full((LANES,), a, jnp.int32)])

        @plsc.parallel_loop(0, VECS, unroll=4)
        def _zero(i):
            acc_v[pl.ds(i * LANES, LANES)] = jnp.zeros((LANES,), jnp.float32)

        def elem_body(e, _):
            pltpu.sync_copy(rf_hbm.at[a, e, :], rf_v)
            pltpu.sync_copy(drx_hbm.at[e, pl.ds(base, PIX_PER_TILE)], drx_v)
            pltpu.sync_copy(apod_hbm.at[e, pl.ds(base, PIX_PER_TILE)], apod_v)

            @plsc.parallel_loop(0, VECS, unroll=4)
            def _interp(i):
                sl = pl.ds(i * LANES, LANES)
                pos = (dtx_v[sl] + drx_v[sl] - t0_b) * FS
                i0 = pos.astype(jnp.int32)
                fr = pos - i0.astype(jnp.float32)
                v0 = plsc.load_gather(rf_v, [i0])
                v1 = plsc.load_gather(rf_v, [i0 + 1])
                plsc.addupdate(acc_v.at[sl], (v0 + fr * (v1 - v0)) * apod_v[sl])

            return ()

        lax.fori_loop(0, N_ELEMENTS, elem_body, (), unroll=False)
        pltpu.sync_copy(acc_v, out_hbm.at[a, pl.ds(base, PIX_PER_TILE)])
        return ()

    lax.fori_loop(0, N_ANGLES, angle_body, (), unroll=False)


@jax.jit
def kernel(d_tx, d_rx, apod, rf, t0):
    dtx2 = d_tx.reshape(N_ANGLES, NPIX)
    drx2 = d_rx.reshape(N_ELEMENTS, NPIX)
    apod2 = apod.reshape(N_ELEMENTS, NPIX)
    rf_pad = jnp.pad(rf, ((0, 0), (0, 0), (0, S_PAD - N_SAMP)))

    mesh = plsc.VectorSubcoreMesh(core_axis_name="c", subcore_axis_name="s")
    out = pl.kernel(
        _das_body,
        out_type=jax.ShapeDtypeStruct((N_ANGLES, NPIX), jnp.float32),
        mesh=mesh,
        compiler_params=pltpu.CompilerParams(needs_layout_passes=False),
        scratch_types=[
            pltpu.VMEM((PIX_PER_TILE,), jnp.float32),
            pltpu.VMEM((PIX_PER_TILE,), jnp.float32),
            pltpu.VMEM((PIX_PER_TILE,), jnp.float32),
            pltpu.VMEM((S_PAD,), jnp.float32),
            pltpu.VMEM((PIX_PER_TILE,), jnp.float32),
            pltpu.VMEM((N_ANGLES,), jnp.float32),
        ],
    )(dtx2, drx2, apod2, rf_pad, t0)
    return out.reshape(N_ANGLES, NZ, NX)


# speedup vs baseline: 3168.0169x; 3.3675x over previous
import jax
import jax.numpy as jnp
from jax import lax
from jax.experimental import pallas as pl
from jax.experimental.pallas import tpu as pltpu
from jax.experimental.pallas import tpu_sc as plsc

N_ANGLES, N_ELEMENTS, N_SAMP = 16, 128, 2048
NZ, NX = 512, 256
NPIX = NZ * NX
FS = float(N_SAMP) / 2.0

N_TILES = 32
PIX_PER_TILE = NPIX // N_TILES
S_PAD = N_SAMP + 128
LANES = 16
VECS = PIX_PER_TILE // LANES
G = 4
N_GROUPS = N_ANGLES // G


def _das_body(dtx_hbm, drx_hbm, apod_hbm, rf_hbm, t0_hbm, out_hbm,
              dtx_v, acc_v, rf_v, drx_v, apod_v, t0_v, sem0, sem1):
    wid = lax.axis_index("c") * 16 + lax.axis_index("s")
    base = wid * PIX_PER_TILE
    sems = (sem0, sem1)

    pltpu.sync_copy(t0_hbm, t0_v)

    def start_elem(g, e, b):
        pltpu.async_copy(rf_hbm.at[e, pl.ds(g * (G * S_PAD), G * S_PAD)],
                         rf_v.at[pl.ds(b * G * S_PAD, G * S_PAD)], sems[b])
        pltpu.async_copy(drx_hbm.at[e, pl.ds(base, PIX_PER_TILE)],
                         drx_v.at[b], sems[b])
        pltpu.async_copy(apod_hbm.at[e, pl.ds(base, PIX_PER_TILE)],
                         apod_v.at[b], sems[b])

    def wait_elem(b):
        pltpu.make_async_copy(rf_hbm.at[0, pl.ds(0, G * S_PAD)],
                              rf_v.at[pl.ds(b * G * S_PAD, G * S_PAD)],
                              sems[b]).wait()
        pltpu.make_async_copy(drx_hbm.at[0, pl.ds(0, PIX_PER_TILE)],
                              drx_v.at[b], sems[b]).wait()
        pltpu.make_async_copy(apod_hbm.at[0, pl.ds(0, PIX_PER_TILE)],
                              apod_v.at[b], sems[b]).wait()

    def group_body(g, _):
        for a in range(G):
            pltpu.sync_copy(
                dtx_hbm.at[g * G + a, pl.ds(base, PIX_PER_TILE)], dtx_v.at[a])
        t0_bs = [
            plsc.load_gather(t0_v, [jnp.full((LANES,), g * G + a, jnp.int32)])
            for a in range(G)
        ]

        @plsc.parallel_loop(0, VECS, unroll=4)
        def _zero(i):
            sl = pl.ds(i * LANES, LANES)
            for a in range(G):
                acc_v[a, sl] = jnp.zeros((LANES,), jnp.float32)

        start_elem(g, 0, 0)
        start_elem(g, 1, 1)

        def elem_pair(j, _):
            for b in (0, 1):
                e = j * 2 + b
                wait_elem(b)

                @plsc.parallel_loop(0, VECS, unroll=4)
                def _interp(i):
                    sl = pl.ds(i * LANES, LANES)
                    drx = drx_v[b, sl]
                    w = apod_v[b, sl]
                    for a in range(G):
                        pos = (dtx_v[a, sl] + drx - t0_bs[a]) * FS
                        i0 = pos.astype(jnp.int32)
                        fr = pos - i0.astype(jnp.float32)
                        i0 = i0 + ((b * G + a) * S_PAD)
                        v0 = plsc.load_gather(rf_v, [i0])
                        v1 = plsc.load_gather(rf_v, [i0 + 1])
                        plsc.addupdate(acc_v.at[a, sl],
                                       (v0 + fr * (v1 - v0)) * w)

                @pl.when(e + 2 < N_ELEMENTS)
                def _():
                    start_elem(g, e + 2, b)

            return ()

        lax.fori_loop(0, N_ELEMENTS // 2, elem_pair, (), unroll=False)

        for a in range(G):
            pltpu.sync_copy(acc_v.at[a],
                            out_hbm.at[g * G + a, pl.ds(base, PIX_PER_TILE)])
        return ()

    lax.fori_loop(0, N_GROUPS, group_body, (), unroll=False)


@jax.jit
def kernel(d_tx, d_rx, apod, rf, t0):
    dtx2 = d_tx.reshape(N_ANGLES, NPIX)
    drx2 = d_rx.reshape(N_ELEMENTS, NPIX)
    apod2 = apod.reshape(N_ELEMENTS, NPIX)
    rf_pad = jnp.pad(jnp.transpose(rf, (1, 0, 2)),
                     ((0, 0), (0, 0), (0, S_PAD - N_SAMP)))
    rf_pad = rf_pad.reshape(N_ELEMENTS, N_ANGLES * S_PAD)

    mesh = plsc.VectorSubcoreMesh(core_axis_name="c", subcore_axis_name="s")
    out = pl.kernel(
        _das_body,
        out_type=jax.ShapeDtypeStruct((N_ANGLES, NPIX), jnp.float32),
        mesh=mesh,
        compiler_params=pltpu.CompilerParams(needs_layout_passes=False),
        scratch_types=[
            pltpu.VMEM((G, PIX_PER_TILE), jnp.float32),
            pltpu.VMEM((G, PIX_PER_TILE), jnp.float32),
            pltpu.VMEM((2 * G * S_PAD,), jnp.float32),
            pltpu.VMEM((2, PIX_PER_TILE), jnp.float32),
            pltpu.VMEM((2, PIX_PER_TILE), jnp.float32),
            pltpu.VMEM((N_ANGLES,), jnp.float32),
            pltpu.SemaphoreType.DMA,
            pltpu.SemaphoreType.DMA,
        ],
    )(dtx2, drx2, apod2, rf_pad, t0)
    return out.reshape(N_ANGLES, NZ, NX)


# speedup vs baseline: 3411.2409x; 1.0768x over previous
import jax
import jax.numpy as jnp
from jax import lax
from jax.experimental import pallas as pl
from jax.experimental.pallas import tpu as pltpu
from jax.experimental.pallas import tpu_sc as plsc

N_ANGLES, N_ELEMENTS, N_SAMP = 16, 128, 2048
NZ, NX = 512, 256
NPIX = NZ * NX
FS = float(N_SAMP) / 2.0

N_TILES = 32
PIX_PER_TILE = NPIX // N_TILES
S_PAD = N_SAMP + 32
LANES = 16
VECS = PIX_PER_TILE // LANES
G = 8
N_GROUPS = N_ANGLES // G


def _das_body(dtx_hbm, drx_hbm, apod_hbm, rf_hbm, t0_hbm, out_hbm,
              dtx_v, acc_v, rf0_v, rf1_v, drx_v, apod_v, t0_v, sem0, sem1):
    wid = lax.axis_index("c") * 16 + lax.axis_index("s")
    base = wid * PIX_PER_TILE
    sems = (sem0, sem1)
    rfs = (rf0_v, rf1_v)

    pltpu.sync_copy(t0_hbm, t0_v)

    def start_elem(g, e, b):
        pltpu.async_copy(rf_hbm.at[e, pl.ds(g * (G * S_PAD), G * S_PAD)],
                         rfs[b], sems[b])
        pltpu.async_copy(drx_hbm.at[e, pl.ds(base, PIX_PER_TILE)],
                         drx_v.at[b], sems[b])
        pltpu.async_copy(apod_hbm.at[e, pl.ds(base, PIX_PER_TILE)],
                         apod_v.at[b], sems[b])

    def wait_elem(b):
        pltpu.make_async_copy(rf_hbm.at[0, pl.ds(0, G * S_PAD)], rfs[b],
                              sems[b]).wait()
        pltpu.make_async_copy(drx_hbm.at[0, pl.ds(0, PIX_PER_TILE)],
                              drx_v.at[b], sems[b]).wait()
        pltpu.make_async_copy(apod_hbm.at[0, pl.ds(0, PIX_PER_TILE)],
                              apod_v.at[b], sems[b]).wait()

    def group_body(g, _):
        for a in range(G):
            pltpu.sync_copy(
                dtx_hbm.at[g * G + a, pl.ds(base, PIX_PER_TILE)], dtx_v.at[a])

        for a in range(G):
            t0_b = plsc.load_gather(
                t0_v, [jnp.full((LANES,), g * G + a, jnp.int32)])
            cvec = float(a * S_PAD) - t0_b * FS

            @plsc.parallel_loop(0, VECS, unroll=4)
            def _stage(i):
                sl = pl.ds(i * LANES, LANES)
                dtx_v[a, sl] = dtx_v[a, sl] * FS + cvec
                acc_v[a, sl] = jnp.zeros((LANES,), jnp.float32)

        start_elem(g, 0, 0)
        start_elem(g, 1, 1)

        def elem_pair(j, _):
            for b in (0, 1):
                e = j * 2 + b
                wait_elem(b)
                rfb = rfs[b]

                @plsc.parallel_loop(0, VECS, unroll=8)
                def _interp(i):
                    sl = pl.ds(i * LANES, LANES)
                    drxs = drx_v[b, sl] * FS
                    w = apod_v[b, sl]
                    for a in range(G):
                        pos = dtx_v[a, sl] + drxs
                        i0 = pos.astype(jnp.int32)
                        fr = pos - i0.astype(jnp.float32)
                        v0 = plsc.load_gather(rfb, [i0])
                        v1 = plsc.load_gather(rfb, [i0 + 1])
                        plsc.addupdate(acc_v.at[a, sl],
                                       (v0 + fr * (v1 - v0)) * w)

                @pl.when(e + 2 < N_ELEMENTS)
                def _():
                    start_elem(g, e + 2, b)

            return ()

        lax.fori_loop(0, N_ELEMENTS // 2, elem_pair, (), unroll=False)

        for a in range(G):
            pltpu.sync_copy(acc_v.at[a],
                            out_hbm.at[g * G + a, pl.ds(base, PIX_PER_TILE)])
        return ()

    lax.fori_loop(0, N_GROUPS, group_body, (), unroll=False)


@jax.jit
def kernel(d_tx, d_rx, apod, rf, t0):
    dtx2 = d_tx.reshape(N_ANGLES, NPIX)
    drx2 = d_rx.reshape(N_ELEMENTS, NPIX)
    apod2 = apod.reshape(N_ELEMENTS, NPIX)
    rf_pad = jnp.pad(jnp.transpose(rf, (1, 0, 2)),
                     ((0, 0), (0, 0), (0, S_PAD - N_SAMP)))
    rf_pad = rf_pad.reshape(N_ELEMENTS, N_ANGLES * S_PAD)

    mesh = plsc.VectorSubcoreMesh(core_axis_name="c", subcore_axis_name="s")
    out = pl.kernel(
        _das_body,
        out_type=jax.ShapeDtypeStruct((N_ANGLES, NPIX), jnp.float32),
        mesh=mesh,
        compiler_params=pltpu.CompilerParams(needs_layout_passes=False),
        scratch_types=[
            pltpu.VMEM((G, PIX_PER_TILE), jnp.float32),
            pltpu.VMEM((G, PIX_PER_TILE), jnp.float32),
            pltpu.VMEM((G * S_PAD,), jnp.float32),
            pltpu.VMEM((G * S_PAD,), jnp.float32),
            pltpu.VMEM((2, PIX_PER_TILE), jnp.float32),
            pltpu.VMEM((2, PIX_PER_TILE), jnp.float32),
            pltpu.VMEM((N_ANGLES,), jnp.float32),
            pltpu.SemaphoreType.DMA,
            pltpu.SemaphoreType.DMA,
        ],
    )(dtx2, drx2, apod2, rf_pad, t0)
    return out.reshape(N_ANGLES, NZ, NX)
